# baseline (speedup 1.0000x reference)
import functools

import jax
import jax.numpy as jnp
from jax import lax
from jax.experimental import pallas as pl
from jax.experimental.pallas import tpu as pltpu
from jax.experimental.pallas import tpu_sc as plsc

_PH = 7
_PW = 7
_SCALE = 7.0
_L = 16
_NC = 2
_NS = 16


def _splat(v):
    return jnp.full((_L,), v, dtype=jnp.int32)


def _prep_taps(t, size):
    valid = (t >= -1.0) & (t <= float(size))
    t0 = jnp.minimum(jnp.maximum(t, 0.0), float(size))
    tl = t0.astype(jnp.int32)
    cond = tl >= size - 1
    lo = jnp.minimum(tl, size - 1)
    hi = jnp.where(cond, size - 1, tl + 1)
    frac = jnp.where(cond, 0.0, t0 - lo.astype(jnp.float32))
    vf = jnp.where(valid, 1.0, 0.0)
    return lo, hi, (1.0 - frac) * vf, frac * vf


@functools.lru_cache(maxsize=None)
def _make_sc_kernel(N, C, H, W):
    NPTS = _PH * _PW
    HW = H * W
    NWORK = _NC * _NS
    RPW = -(-N // NWORK)
    RPW += RPW % 2
    NGRP = -(-NPTS // _L)
    NP = N + _L

    mesh = plsc.VectorSubcoreMesh(core_axis_name="c", subcore_axis_name="s")

    assert N % 8 == 0 and C % 128 == 0
    NROWS = _PH * _PW * (N // 8) * (C // 128) * 8
    RPR = NPTS * (C // 128)
    PSTRIDE = (N // 8) * (C // 128) * 8

    @functools.partial(
        pl.kernel,
        out_type=jax.ShapeDtypeStruct((NROWS, 128), jnp.float32),
        mesh=mesh,
        scratch_types=[
            pltpu.VMEM((HW, C), jnp.float32),
            pltpu.VMEM((4, NP), jnp.float32),
            pltpu.VMEM((2, RPR, 128), jnp.float32),
            pltpu.VMEM((2, RPR), jnp.int32),
            pltpu.VMEM((4, _L), jnp.int32),
            pltpu.VMEM((4, _L), jnp.float32),
            pltpu.VMEM((4, NGRP * _L), jnp.int32),
            pltpu.VMEM((4, NGRP * _L), jnp.float32),
            pltpu.SemaphoreType.DMA,
            pltpu.SemaphoreType.DMA,
        ],
        compiler_params=pltpu.CompilerParams(use_tc_tiling_on_sc=False,
                                             needs_layout_passes=False),
    )
    def sc_kernel(table_hbm, sw_hbm, sh_hbm, bw_hbm, bh_hbm, out_hbm, tbl_v,
                  s4, obuf, oidx, idx_s, w_s, r2_s, w2_s, sem0, sem1):
        sems = (sem0, sem1)
        wid = lax.axis_index("s") * _NC + lax.axis_index("c")
        base = wid * RPW
        pltpu.sync_copy(table_hbm, tbl_v)
        for t, src in enumerate((sw_hbm, sh_hbm, bw_hbm, bh_hbm)):
            pltpu.sync_copy(src, s4.at[t, pl.ds(0, N)])
        i16 = jnp.arange(_L, dtype=jnp.int32)
        f16 = i16.astype(jnp.float32)
        lane4 = jnp.minimum(i16, 3)

        def compute_roi(groi, slot):
            rv = plsc.load_gather(s4, [lane4, _splat(groi)])
            sw = jnp.full((_L,), rv[0], dtype=jnp.float32)
            sh = jnp.full((_L,), rv[1], dtype=jnp.float32)
            bw = jnp.full((_L,), rv[2], dtype=jnp.float32)
            bh = jnp.full((_L,), rv[3], dtype=jnp.float32)
            y = sh + (f16 + 0.5) * bh
            x = sw + (f16 + 0.5) * bw
            ylo, yhi, wyl, wyh = _prep_taps(y, H)
            xlo, xhi, wxl, wxh = _prep_taps(x, W)
            idx_s[0, :] = ylo * W
            idx_s[1, :] = yhi * W
            idx_s[2, :] = xlo
            idx_s[3, :] = xhi
            w_s[0, :] = wyl
            w_s[1, :] = wyh
            w_s[2, :] = wxl
            w_s[3, :] = wxh
            for g in range(NGRP):
                p = jnp.minimum(i16 + _L * g, NPTS - 1)
                ph = p // _PW
                pw = p % _PW
                gy = [plsc.load_gather(idx_s, [_splat(t), ph]) for t in (0, 1)]
                gx = [plsc.load_gather(idx_s, [_splat(t), pw]) for t in (2, 3)]
                wy = [plsc.load_gather(w_s, [_splat(t), ph]) for t in (0, 1)]
                wx = [plsc.load_gather(w_s, [_splat(t), pw]) for t in (2, 3)]
                for t, (a, b) in enumerate(((0, 0), (0, 1), (1, 0), (1, 1))):
                    r2_s[t, pl.ds(_L * g, _L)] = gy[a] + gx[b]
                    w2_s[t, pl.ds(_L * g, _L)] = wy[a] * wx[b]

            base_n = (groi // 8) * ((C // 128) * 8) + groi % 8
            bn_v = _splat(base_n)
            for m in range(-(-RPR // _L)):
                r = i16 + _L * m
                val = (r >> 1) * PSTRIDE + (r & 1) * 8 + bn_v
                if _L * (m + 1) <= RPR:
                    oidx[slot, pl.ds(_L * m, _L)] = val
                else:
                    plsc.store_scatter(oidx, [_splat(slot), _L * m + i16],
                                       val, mask=i16 < (RPR - _L * m))

            @plsc.parallel_loop(0, NPTS, unroll=3)
            def pbody(p):
                pd = pl.ds(p, _L)
                r0 = r2_s[0, pd][0]
                r1 = r2_s[1, pd][0]
                r2 = r2_s[2, pd][0]
                r3 = r2_s[3, pd][0]
                p_v = _splat(p)
                w0 = plsc.load_gather(w2_s, [_splat(0), p_v])
                w1 = plsc.load_gather(w2_s, [_splat(1), p_v])
                w2 = plsc.load_gather(w2_s, [_splat(2), p_v])
                w3 = plsc.load_gather(w2_s, [_splat(3), p_v])
                row2 = p * (C // 128)
                for k in range(C // _L):
                    ck = pl.ds(_L * k, _L)
                    acc = (w0 * tbl_v[r0, ck] + w1 * tbl_v[r1, ck]
                           + w2 * tbl_v[r2, ck] + w3 * tbl_v[r3, ck])
                    obuf[slot, row2 + k // 8, pl.ds((k % 8) * _L, _L)] = acc

        def pair_body(rr, _):
            for b in range(2):
                groi = jnp.minimum(base + 2 * rr + b, N - 1)

                @pl.when(rr > 0)
                def _wait():
                    pltpu.make_async_copy(obuf.at[b], out_hbm.at[oidx.at[b]],
                                          sems[b]).wait()

                compute_roi(groi, b)
                pltpu.async_copy(obuf.at[b], out_hbm.at[oidx.at[b]], sems[b])
            return 0

        lax.fori_loop(0, RPW // 2, pair_body, 0)
        for b in range(2):
            pltpu.make_async_copy(obuf.at[b], out_hbm.at[oidx.at[b]],
                                  sems[b]).wait()

    return sc_kernel


def kernel(features, rois):
    _, C, H, W = features.shape
    N = rois.shape[0]
    table = jnp.transpose(features.reshape(C, H * W))
    sw = rois[:, 1] * _SCALE
    sh = rois[:, 2] * _SCALE
    ew = rois[:, 3] * _SCALE
    eh = rois[:, 4] * _SCALE
    bw = jnp.maximum(ew - sw, 1.0) / _PW
    bh = jnp.maximum(eh - sh, 1.0) / _PH
    out = _make_sc_kernel(N, C, H, W)(table, sw, sh, bw, bh)
    out = out.reshape(_PH, _PW, N // 8, C // 128, 8, 128)
    out = jnp.transpose(out, (2, 4, 3, 5, 0, 1))
    return out.reshape(N, C, _PH, _PW)


# speedup vs baseline: 1.0639x; 1.0639x over previous
import functools

import jax
import jax.numpy as jnp
from jax import lax
from jax.experimental import pallas as pl
from jax.experimental.pallas import tpu as pltpu
from jax.experimental.pallas import tpu_sc as plsc

_PH = 7
_PW = 7
_SCALE = 7.0
_L = 16
_NC = 2
_NS = 16


def _splat(v):
    return jnp.full((_L,), v, dtype=jnp.int32)


def _prep_taps(t, size):
    valid = (t >= -1.0) & (t <= float(size))
    t0 = jnp.minimum(jnp.maximum(t, 0.0), float(size))
    tl = t0.astype(jnp.int32)
    cond = tl >= size - 1
    lo = jnp.minimum(tl, size - 1)
    hi = jnp.where(cond, size - 1, tl + 1)
    frac = jnp.where(cond, 0.0, t0 - lo.astype(jnp.float32))
    vf = jnp.where(valid, 1.0, 0.0)
    return lo, hi, (1.0 - frac) * vf, frac * vf


@functools.lru_cache(maxsize=None)
def _make_sc_kernel(N, C, H, W):
    NPTS = _PH * _PW
    HW = H * W
    NWORK = _NC * _NS
    RPW = -(-N // NWORK)
    RPW += RPW % 2
    NGRP = -(-NPTS // _L)
    NP = N + _L

    mesh = plsc.VectorSubcoreMesh(core_axis_name="c", subcore_axis_name="s")

    assert N % 8 == 0 and C % 128 == 0
    NROWS = _PH * _PW * (N // 8) * (C // 128) * 8
    RPR = NPTS * (C // 128)
    PSTRIDE = (N // 8) * (C // 128) * 8

    @functools.partial(
        pl.kernel,
        out_type=jax.ShapeDtypeStruct((NROWS, 128), jnp.float32),
        mesh=mesh,
        scratch_types=[
            pltpu.VMEM((HW, C), jnp.float32),
            pltpu.VMEM((4, NP), jnp.float32),
            pltpu.VMEM((2, RPR, 128), jnp.float32),
            pltpu.VMEM((2, RPR), jnp.int32),
            pltpu.VMEM((4, _L), jnp.int32),
            pltpu.VMEM((4, _L), jnp.float32),
            pltpu.VMEM((NGRP * _L * 4,), jnp.int32),
            pltpu.VMEM((NGRP * _L * 4,), jnp.float32),
            pltpu.SemaphoreType.DMA,
            pltpu.SemaphoreType.DMA,
        ],
        compiler_params=pltpu.CompilerParams(use_tc_tiling_on_sc=False,
                                             needs_layout_passes=False),
    )
    def sc_kernel(table_hbm, sw_hbm, sh_hbm, bw_hbm, bh_hbm, out_hbm, tbl_v,
                  s4, obuf, oidx, idx_s, w_s, r2_s, w2_s, sem0, sem1):
        sems = (sem0, sem1)
        wid = lax.axis_index("s") * _NC + lax.axis_index("c")
        base = wid * RPW
        pltpu.sync_copy(table_hbm, tbl_v)
        for t, src in enumerate((sw_hbm, sh_hbm, bw_hbm, bh_hbm)):
            pltpu.sync_copy(src, s4.at[t, pl.ds(0, N)])
        i16 = jnp.arange(_L, dtype=jnp.int32)
        f16 = i16.astype(jnp.float32)
        lane4 = jnp.minimum(i16, 3)

        def compute_roi(groi, slot):
            rv = plsc.load_gather(s4, [lane4, _splat(groi)])
            sw = jnp.full((_L,), rv[0], dtype=jnp.float32)
            sh = jnp.full((_L,), rv[1], dtype=jnp.float32)
            bw = jnp.full((_L,), rv[2], dtype=jnp.float32)
            bh = jnp.full((_L,), rv[3], dtype=jnp.float32)
            y = sh + (f16 + 0.5) * bh
            x = sw + (f16 + 0.5) * bw
            ylo, yhi, wyl, wyh = _prep_taps(y, H)
            xlo, xhi, wxl, wxh = _prep_taps(x, W)
            idx_s[0, :] = ylo * W
            idx_s[1, :] = yhi * W
            idx_s[2, :] = xlo
            idx_s[3, :] = xhi
            w_s[0, :] = wyl
            w_s[1, :] = wyh
            w_s[2, :] = wxl
            w_s[3, :] = wxh
            for g in range(NGRP):
                p = jnp.minimum(i16 + _L * g, NPTS - 1)
                ph = p // _PW
                pw = p % _PW
                gy = [plsc.load_gather(idx_s, [_splat(t), ph]) for t in (0, 1)]
                gx = [plsc.load_gather(idx_s, [_splat(t), pw]) for t in (2, 3)]
                wy = [plsc.load_gather(w_s, [_splat(t), ph]) for t in (0, 1)]
                wx = [plsc.load_gather(w_s, [_splat(t), pw]) for t in (2, 3)]
                i16_4 = i16 * 4
                for t, (a, b) in enumerate(((0, 0), (0, 1), (1, 0), (1, 1))):
                    plsc.store_scatter(r2_s, [i16_4 + (4 * _L * g + t)],
                                       gy[a] + gx[b])
                    plsc.store_scatter(w2_s, [i16_4 + (4 * _L * g + t)],
                                       wy[a] * wx[b])

            base_n = (groi // 8) * ((C // 128) * 8) + groi % 8
            bn_v = _splat(base_n)
            for m in range(-(-RPR // _L)):
                r = i16 + _L * m
                val = (r >> 1) * PSTRIDE + (r & 1) * 8 + bn_v
                if _L * (m + 1) <= RPR:
                    oidx[slot, pl.ds(_L * m, _L)] = val
                else:
                    plsc.store_scatter(oidx, [_splat(slot), _L * m + i16],
                                       val, mask=i16 < (RPR - _L * m))

            @plsc.parallel_loop(0, NPTS, unroll=3)
            def pbody(p):
                pd = pl.ds(p * 4, _L)
                rv = r2_s[pd]
                wv = w2_s[pd]
                r0 = rv[0]
                r1 = rv[1]
                r2 = rv[2]
                r3 = rv[3]
                w0 = jnp.full((_L,), wv[0], dtype=jnp.float32)
                w1 = jnp.full((_L,), wv[1], dtype=jnp.float32)
                w2 = jnp.full((_L,), wv[2], dtype=jnp.float32)
                w3 = jnp.full((_L,), wv[3], dtype=jnp.float32)
                row2 = p * (C // 128)
                for k in range(C // _L):
                    ck = pl.ds(_L * k, _L)
                    acc = (w0 * tbl_v[r0, ck] + w1 * tbl_v[r1, ck]
                           + w2 * tbl_v[r2, ck] + w3 * tbl_v[r3, ck])
                    obuf[slot, row2 + k // 8, pl.ds((k % 8) * _L, _L)] = acc

        def pair_body(rr, _):
            for b in range(2):
                groi = jnp.minimum(base + 2 * rr + b, N - 1)

                @pl.when(rr > 0)
                def _wait():
                    pltpu.make_async_copy(obuf.at[b], out_hbm.at[oidx.at[b]],
                                          sems[b]).wait()

                compute_roi(groi, b)
                pltpu.async_copy(obuf.at[b], out_hbm.at[oidx.at[b]], sems[b])
            return 0

        lax.fori_loop(0, RPW // 2, pair_body, 0)
        for b in range(2):
            pltpu.make_async_copy(obuf.at[b], out_hbm.at[oidx.at[b]],
                                  sems[b]).wait()

    return sc_kernel


def kernel(features, rois):
    _, C, H, W = features.shape
    N = rois.shape[0]
    table = jnp.transpose(features.reshape(C, H * W))
    sw = rois[:, 1] * _SCALE
    sh = rois[:, 2] * _SCALE
    ew = rois[:, 3] * _SCALE
    eh = rois[:, 4] * _SCALE
    bw = jnp.maximum(ew - sw, 1.0) / _PW
    bh = jnp.maximum(eh - sh, 1.0) / _PH
    out = _make_sc_kernel(N, C, H, W)(table, sw, sh, bw, bh)
    out = out.reshape(_PH, _PW, N // 8, C // 128, 8, 128)
    out = jnp.transpose(out, (2, 4, 3, 5, 0, 1))
    return out.reshape(N, C, _PH, _PW)


# speedup vs baseline: 1.1001x; 1.0341x over previous
import functools

import jax
import jax.numpy as jnp
from jax import lax
from jax.experimental import pallas as pl
from jax.experimental.pallas import tpu as pltpu
from jax.experimental.pallas import tpu_sc as plsc

_PH = 7
_PW = 7
_SCALE = 7.0
_L = 16
_NC = 2
_NS = 16


def _splat(v):
    return jnp.full((_L,), v, dtype=jnp.int32)


def _prep_taps(t, size):
    valid = (t >= -1.0) & (t <= float(size))
    t0 = jnp.minimum(jnp.maximum(t, 0.0), float(size))
    tl = t0.astype(jnp.int32)
    cond = tl >= size - 1
    lo = jnp.minimum(tl, size - 1)
    hi = jnp.where(cond, size - 1, tl + 1)
    frac = jnp.where(cond, 0.0, t0 - lo.astype(jnp.float32))
    vf = jnp.where(valid, 1.0, 0.0)
    return lo, hi, (1.0 - frac) * vf, frac * vf


@functools.lru_cache(maxsize=None)
def _make_sc_kernel(N, C, H, W):
    NPTS = _PH * _PW
    HW = H * W
    NWORK = _NC * _NS
    RPW = -(-N // NWORK)
    RPW += RPW % 2
    NGRP = -(-NPTS // _L)
    NP = N + _L

    mesh = plsc.VectorSubcoreMesh(core_axis_name="c", subcore_axis_name="s")

    assert N % 8 == 0 and C % 128 == 0
    NROWS = _PH * _PW * (N // 8) * (C // 128) * 8
    RPR = NPTS * (C // 128)
    PSTRIDE = (N // 8) * (C // 128) * 8

    @functools.partial(
        pl.kernel,
        out_type=jax.ShapeDtypeStruct((NROWS, 128), jnp.float32),
        mesh=mesh,
        scratch_types=[
            pltpu.VMEM((HW, C), jnp.float32),
            pltpu.VMEM((4, NP), jnp.float32),
            pltpu.VMEM((2, RPR, 128), jnp.float32),
            pltpu.VMEM((2, RPR), jnp.int32),
            pltpu.VMEM((4, _L), jnp.int32),
            pltpu.VMEM((4, _L), jnp.float32),
            pltpu.VMEM((NGRP * _L * 4,), jnp.int32),
            pltpu.VMEM((NGRP * _L * 4,), jnp.float32),
            pltpu.SemaphoreType.DMA,
            pltpu.SemaphoreType.DMA,
        ],
        compiler_params=pltpu.CompilerParams(use_tc_tiling_on_sc=False,
                                             needs_layout_passes=False),
    )
    def sc_kernel(table_hbm, sw_hbm, sh_hbm, bw_hbm, bh_hbm, out_hbm, tbl_v,
                  s4, obuf, oidx, idx_s, w_s, r2_s, w2_s, sem0, sem1):
        sems = (sem0, sem1)
        wid = lax.axis_index("s") * _NC + lax.axis_index("c")
        base = wid * RPW
        stage = [pltpu.async_copy(table_hbm, tbl_v, sem0)]
        for t, src in enumerate((sw_hbm, sh_hbm, bw_hbm, bh_hbm)):
            stage.append(pltpu.async_copy(src, s4.at[t, pl.ds(0, N)], sem1))
        for d in stage:
            d.wait()
        i16 = jnp.arange(_L, dtype=jnp.int32)
        f16 = i16.astype(jnp.float32)
        lane4 = jnp.minimum(i16, 3)

        def compute_roi(groi, slot):
            rv = plsc.load_gather(s4, [lane4, _splat(groi)])
            sw = jnp.full((_L,), rv[0], dtype=jnp.float32)
            sh = jnp.full((_L,), rv[1], dtype=jnp.float32)
            bw = jnp.full((_L,), rv[2], dtype=jnp.float32)
            bh = jnp.full((_L,), rv[3], dtype=jnp.float32)
            y = sh + (f16 + 0.5) * bh
            x = sw + (f16 + 0.5) * bw
            ylo, yhi, wyl, wyh = _prep_taps(y, H)
            xlo, xhi, wxl, wxh = _prep_taps(x, W)
            idx_s[0, :] = ylo * W
            idx_s[1, :] = yhi * W
            idx_s[2, :] = xlo
            idx_s[3, :] = xhi
            w_s[0, :] = wyl
            w_s[1, :] = wyh
            w_s[2, :] = wxl
            w_s[3, :] = wxh
            for g in range(NGRP):
                p = jnp.minimum(i16 + _L * g, NPTS - 1)
                ph = p // _PW
                pw = p % _PW
                gy = [plsc.load_gather(idx_s, [_splat(t), ph]) for t in (0, 1)]
                gx = [plsc.load_gather(idx_s, [_splat(t), pw]) for t in (2, 3)]
                wy = [plsc.load_gather(w_s, [_splat(t), ph]) for t in (0, 1)]
                wx = [plsc.load_gather(w_s, [_splat(t), pw]) for t in (2, 3)]
                i16_4 = i16 * 4
                for t, (a, b) in enumerate(((0, 0), (0, 1), (1, 0), (1, 1))):
                    plsc.store_scatter(r2_s, [i16_4 + (4 * _L * g + t)],
                                       gy[a] + gx[b])
                    plsc.store_scatter(w2_s, [i16_4 + (4 * _L * g + t)],
                                       wy[a] * wx[b])

            base_n = (groi // 8) * ((C // 128) * 8) + groi % 8
            bn_v = _splat(base_n)
            for m in range(-(-RPR // _L)):
                r = i16 + _L * m
                val = (r >> 1) * PSTRIDE + (r & 1) * 8 + bn_v
                if _L * (m + 1) <= RPR:
                    oidx[slot, pl.ds(_L * m, _L)] = val
                else:
                    plsc.store_scatter(oidx, [_splat(slot), _L * m + i16],
                                       val, mask=i16 < (RPR - _L * m))

            @plsc.parallel_loop(0, NPTS, unroll=3)
            def pbody(p):
                pd = pl.ds(p * 4, _L)
                rv = r2_s[pd]
                wv = w2_s[pd]
                r0 = rv[0]
                r1 = rv[1]
                r2 = rv[2]
                r3 = rv[3]
                w0 = jnp.full((_L,), wv[0], dtype=jnp.float32)
                w1 = jnp.full((_L,), wv[1], dtype=jnp.float32)
                w2 = jnp.full((_L,), wv[2], dtype=jnp.float32)
                w3 = jnp.full((_L,), wv[3], dtype=jnp.float32)
                row2 = p * (C // 128)
                for k in range(C // _L):
                    ck = pl.ds(_L * k, _L)
                    acc = (w0 * tbl_v[r0, ck] + w1 * tbl_v[r1, ck]
                           + w2 * tbl_v[r2, ck] + w3 * tbl_v[r3, ck])
                    obuf[slot, row2 + k // 8, pl.ds((k % 8) * _L, _L)] = acc

        def pair_body(rr, _):
            for b in range(2):
                groi = jnp.minimum(base + 2 * rr + b, N - 1)

                @pl.when(rr > 0)
                def _wait():
                    pltpu.make_async_copy(obuf.at[b], out_hbm.at[oidx.at[b]],
                                          sems[b]).wait()

                compute_roi(groi, b)
                pltpu.async_copy(obuf.at[b], out_hbm.at[oidx.at[b]], sems[b])
            return 0

        lax.fori_loop(0, RPW // 2, pair_body, 0)
        for b in range(2):
            pltpu.make_async_copy(obuf.at[b], out_hbm.at[oidx.at[b]],
                                  sems[b]).wait()

    return sc_kernel


def kernel(features, rois):
    _, C, H, W = features.shape
    N = rois.shape[0]
    table = jnp.transpose(features.reshape(C, H * W))
    sw = rois[:, 1] * _SCALE
    sh = rois[:, 2] * _SCALE
    ew = rois[:, 3] * _SCALE
    eh = rois[:, 4] * _SCALE
    bw = jnp.maximum(ew - sw, 1.0) / _PW
    bh = jnp.maximum(eh - sh, 1.0) / _PH
    out = _make_sc_kernel(N, C, H, W)(table, sw, sh, bw, bh)
    out = out.reshape(_PH, _PW, N // 8, C // 128, 8, 128)
    out = jnp.transpose(out, (2, 4, 3, 5, 0, 1))
    return out.reshape(N, C, _PH, _PW)
